# Initial kernel scaffold; baseline (speedup 1.0000x reference)
import jax
import jax.numpy as jnp
from jax.experimental import pallas as pl


def kernel(diag_codes, proc_codes, prev_med_codes, W_diag, W_proc, W_med, W1, b1, W2, b2):
    raise NotImplementedError("write your pallas kernel here")


# speedup vs baseline: 5.3244x; 5.3244x over previous
import functools

import jax
import jax.numpy as jnp
from jax import lax
from jax.experimental import pallas as pl
from jax.experimental.pallas import tpu as pltpu
from jax.experimental.pallas import tpu_sc as plsc

B = 4096
L = 50
EMBED = 128
LANES = 16
NC = 2
NS = 16
NW = NC * NS
BPW = B // NW
NCHUNK = EMBED // LANES


def _sc_gather_pool(diag_codes, proc_codes, prev_med_codes, W_diag, W_proc, W_med):
    mesh = plsc.VectorSubcoreMesh(core_axis_name="c", subcore_axis_name="s")

    @functools.partial(
        pl.kernel,
        mesh=mesh,
        out_type=jax.ShapeDtypeStruct((B, 3 * EMBED), jnp.float32),
        scratch_types=[
            pltpu.VMEM((BPW, L), jnp.int32),
            pltpu.VMEM((L, EMBED), jnp.float32),
            pltpu.VMEM((BPW, 3 * EMBED), jnp.float32),
            pltpu.SemaphoreType.DMA,
        ],
    )
    def k(diag_hbm, proc_hbm, med_hbm, wd_hbm, wp_hbm, wm_hbm, out_hbm,
          idx_v, rows_v, out_v, sem):
        wid = lax.axis_index("s") * NC + lax.axis_index("c")
        base = wid * BPW

        for t, (codes_hbm, table_hbm) in enumerate(
                ((diag_hbm, wd_hbm), (proc_hbm, wp_hbm), (med_hbm, wm_hbm))):
            pltpu.sync_copy(codes_hbm.at[pl.ds(base, BPW)], idx_v)

            def visit_body(v, _, table_hbm=table_hbm, t=t):
                pltpu.async_copy(table_hbm.at[idx_v.at[v]], rows_v, sem).wait()

                def row_body(r, accs):
                    return tuple(
                        accs[c] + rows_v[r, pl.ds(c * LANES, LANES)]
                        for c in range(NCHUNK))

                accs = lax.fori_loop(
                    0, L, row_body,
                    tuple(jnp.zeros((LANES,), jnp.float32)
                          for _ in range(NCHUNK)))
                for c in range(NCHUNK):
                    out_v[v, pl.ds(t * EMBED + c * LANES, LANES)] = accs[c]
                return 0

            lax.fori_loop(0, BPW, visit_body, 0)

        pltpu.sync_copy(out_v, out_hbm.at[pl.ds(base, BPW)])

    return k(diag_codes, proc_codes, prev_med_codes, W_diag, W_proc, W_med)


def _mlp_body(x_ref, w1_ref, b1_ref, w2_ref, b2_ref, o_ref):
    h = jnp.dot(x_ref[...], w1_ref[...], preferred_element_type=jnp.float32)
    h = jnp.maximum(h + b1_ref[...], 0.0)
    z = jnp.dot(h, w2_ref[...], preferred_element_type=jnp.float32)
    o_ref[...] = jax.nn.sigmoid(z + b2_ref[...])


def _tc_mlp(combined, W1, b1, W2, b2):
    blk = 512
    nout = W2.shape[1]
    return pl.pallas_call(
        _mlp_body,
        grid=(B // blk,),
        in_specs=[
            pl.BlockSpec((blk, 3 * EMBED), lambda i: (i, 0)),
            pl.BlockSpec((3 * EMBED, 64), lambda i: (0, 0)),
            pl.BlockSpec((1, 64), lambda i: (0, 0)),
            pl.BlockSpec((64, nout), lambda i: (0, 0)),
            pl.BlockSpec((1, nout), lambda i: (0, 0)),
        ],
        out_specs=pl.BlockSpec((blk, nout), lambda i: (i, 0)),
        out_shape=jax.ShapeDtypeStruct((B, nout), jnp.float32),
    )(combined, W1, b1.reshape(1, -1), W2, b2.reshape(1, -1))


def kernel(diag_codes, proc_codes, prev_med_codes, W_diag, W_proc, W_med,
           W1, b1, W2, b2):
    combined = _sc_gather_pool(diag_codes, proc_codes, prev_med_codes,
                               W_diag, W_proc, W_med)
    return _tc_mlp(combined, W1, b1, W2, b2)


# speedup vs baseline: 10.7182x; 2.0130x over previous
import functools

import jax
import jax.numpy as jnp
from jax import lax
from jax.experimental import pallas as pl
from jax.experimental.pallas import tpu as pltpu
from jax.experimental.pallas import tpu_sc as plsc

B = 4096
L = 50
EMBED = 128
LANES = 16
NC = 2
NS = 16
NW = NC * NS
BPW = B // NW
PAIRS = BPW // 2
PL2 = 2 * L
NCHUNK = EMBED // LANES


def _sc_gather_pool(diag_codes, proc_codes, prev_med_codes, W_diag, W_proc, W_med):
    mesh = plsc.VectorSubcoreMesh(core_axis_name="c", subcore_axis_name="s")

    @functools.partial(
        pl.kernel,
        mesh=mesh,
        out_type=jax.ShapeDtypeStruct((B, 3 * EMBED), jnp.float32),
        scratch_types=[
            pltpu.VMEM((PAIRS, PL2), jnp.int32),
            pltpu.VMEM((PL2, EMBED), jnp.float32),
            pltpu.VMEM((PL2, EMBED), jnp.float32),
            pltpu.VMEM((BPW, 3 * EMBED), jnp.float32),
            pltpu.SemaphoreType.DMA,
            pltpu.SemaphoreType.DMA,
        ],
    )
    def k(diag_hbm, proc_hbm, med_hbm, wd_hbm, wp_hbm, wm_hbm, out_hbm,
          idx_v, rows0, rows1, out_v, sem0, sem1):
        wid = lax.axis_index("s") * NC + lax.axis_index("c")
        pbase = wid * PAIRS

        def fire(table_hbm, p, rows, sem):
            pltpu.make_async_copy(table_hbm.at[idx_v.at[p]], rows, sem).start()

        for t, (codes_hbm, table_hbm) in enumerate(
                ((diag_hbm, wd_hbm), (proc_hbm, wp_hbm), (med_hbm, wm_hbm))):
            pltpu.sync_copy(codes_hbm.at[pl.ds(pbase, PAIRS)], idx_v)
            fire(table_hbm, 0, rows0, sem0)
            fire(table_hbm, 1, rows1, sem1)

            def pair_body(p2, _, table_hbm=table_hbm, t=t):
                for par, (rows, sem) in enumerate(
                        ((rows0, sem0), (rows1, sem1))):
                    p = p2 * 2 + par
                    pltpu.make_async_copy(
                        table_hbm.at[idx_v.at[0]], rows, sem).wait()

                    def row_body(r, accs, rows=rows):
                        new_a = tuple(
                            accs[c] + rows[r, pl.ds(c * LANES, LANES)]
                            for c in range(NCHUNK))
                        new_b = tuple(
                            accs[NCHUNK + c] + rows[L + r, pl.ds(c * LANES, LANES)]
                            for c in range(NCHUNK))
                        return new_a + new_b

                    accs = lax.fori_loop(
                        0, L, row_body,
                        tuple(jnp.zeros((LANES,), jnp.float32)
                              for _ in range(2 * NCHUNK)))
                    for c in range(NCHUNK):
                        out_v[2 * p, pl.ds(t * EMBED + c * LANES, LANES)] = accs[c]
                        out_v[2 * p + 1, pl.ds(t * EMBED + c * LANES, LANES)] = (
                            accs[NCHUNK + c])

                    @pl.when(p + 2 < PAIRS)
                    def _(table_hbm=table_hbm, p=p, rows=rows, sem=sem):
                        fire(table_hbm, p + 2, rows, sem)
                return 0

            lax.fori_loop(0, PAIRS // 2, pair_body, 0)

        pltpu.sync_copy(out_v, out_hbm.at[pl.ds(wid * BPW, BPW)])

    return k(diag_codes, proc_codes, prev_med_codes, W_diag, W_proc, W_med)


def _mlp_body(x_ref, w1_ref, b1_ref, w2_ref, b2_ref, o_ref):
    h = jnp.dot(x_ref[...], w1_ref[...], preferred_element_type=jnp.float32)
    h = jnp.maximum(h + b1_ref[...], 0.0)
    z = jnp.dot(h, w2_ref[...], preferred_element_type=jnp.float32)
    o_ref[...] = jax.nn.sigmoid(z + b2_ref[...])


def _tc_mlp(combined, W1, b1, W2, b2):
    blk = 512
    nout = W2.shape[1]
    return pl.pallas_call(
        _mlp_body,
        grid=(B // blk,),
        in_specs=[
            pl.BlockSpec((blk, 3 * EMBED), lambda i: (i, 0)),
            pl.BlockSpec((3 * EMBED, 64), lambda i: (0, 0)),
            pl.BlockSpec((1, 64), lambda i: (0, 0)),
            pl.BlockSpec((64, nout), lambda i: (0, 0)),
            pl.BlockSpec((1, nout), lambda i: (0, 0)),
        ],
        out_specs=pl.BlockSpec((blk, nout), lambda i: (i, 0)),
        out_shape=jax.ShapeDtypeStruct((B, nout), jnp.float32),
    )(combined, W1, b1.reshape(1, -1), W2, b2.reshape(1, -1))


def kernel(diag_codes, proc_codes, prev_med_codes, W_diag, W_proc, W_med,
           W1, b1, W2, b2):
    combined = _sc_gather_pool(
        diag_codes.reshape(B // 2, PL2),
        proc_codes.reshape(B // 2, PL2),
        prev_med_codes.reshape(B // 2, PL2),
        W_diag, W_proc, W_med)
    return _tc_mlp(combined, W1, b1, W2, b2)
